# baseline (speedup 1.0000x reference)
import functools

import jax
import jax.numpy as jnp
from jax import lax
from jax.experimental import pallas as pl
from jax.experimental.pallas import tpu as pltpu
from jax.experimental.pallas import tpu_sc as plsc

BATCH = 16384
HIDDEN = 64
LANES = 16
NROWS = 1000000
NC = 2
NS = 16
NW = NC * NS
BPW = BATCH // NW
CHUNK = 128
NCHUNK = BPW // CHUNK


TBLK = 8192
QTR = 30 * TBLK
N2 = NROWS - 3 * QTR
TGRID = -(-N2 // TBLK)


def _pack2(a_ref, b_ref):
    a = a_ref[...].astype(jnp.bfloat16).T
    b = b_ref[...].astype(jnp.bfloat16).T
    ai = lax.bitcast_convert_type(a, jnp.uint16).astype(jnp.uint32)
    bi = lax.bitcast_convert_type(b, jnp.uint16).astype(jnp.uint32)
    return lax.bitcast_convert_type(ai | (bi << jnp.uint32(16)), jnp.int32)


def _tpack_body(t0_ref, t1_ref, t2_ref, t3_ref, o_ref):
    o_ref[:, :HIDDEN] = _pack2(t0_ref, t1_ref)
    o_ref[:, HIDDEN:] = _pack2(t2_ref, t3_ref)


_tpack = pl.pallas_call(
    _tpack_body,
    grid=(TGRID,),
    in_specs=[
        pl.BlockSpec((HIDDEN, TBLK), lambda n: (0, n)),
        pl.BlockSpec((HIDDEN, TBLK), lambda n: (0, n + 30)),
        pl.BlockSpec((HIDDEN, TBLK), lambda n: (0, n + 60)),
        pl.BlockSpec((HIDDEN, TBLK), lambda n: (0, n + 90)),
    ],
    out_specs=pl.BlockSpec((TBLK, 2 * HIDDEN), lambda n: (n, 0)),
    out_shape=jax.ShapeDtypeStruct((N2, 2 * HIDDEN), jnp.int32),
    compiler_params=pltpu.CompilerParams(
        dimension_semantics=("arbitrary",)),
)


_mesh = plsc.VectorSubcoreMesh(core_axis_name="c", subcore_axis_name="s")


@functools.partial(
    pl.kernel,
    mesh=_mesh,
    out_type=jax.ShapeDtypeStruct((BATCH, HIDDEN), jnp.float32),
    scratch_types=[
        pltpu.VMEM((BPW,), jnp.int32),
        pltpu.VMEM((NCHUNK, CHUNK), jnp.int32),
        pltpu.VMEM((CHUNK, 2 * HIDDEN), jnp.int32),
        pltpu.VMEM((CHUNK, 2 * HIDDEN), jnp.int32),
        pltpu.VMEM((CHUNK, HIDDEN), jnp.float32),
        pltpu.VMEM((CHUNK, HIDDEN), jnp.float32),
        pltpu.SemaphoreType.DMA,
        pltpu.SemaphoreType.DMA,
        pltpu.SemaphoreType.DMA,
    ],
    compiler_params=pltpu.CompilerParams(use_tc_tiling_on_sc=True,
                                         needs_layout_passes=False),
)
def _sc_gather(id_hbm, p_hbm, out_hbm, idf_v, gidx_v, rbuf0, rbuf1,
               sel0, sel1, gsem0, gsem1, wsem):
    wid = lax.axis_index("s") * NC + lax.axis_index("c")
    base = wid * BPW
    pltpu.sync_copy(id_hbm.at[wid], idf_v)

    lane_iota = lax.iota(jnp.int32, LANES)

    def quarter(v):
        return ((v >= QTR).astype(jnp.int32) +
                (v >= 2 * QTR).astype(jnp.int32) +
                (v >= 3 * QTR).astype(jnp.int32))

    for j in range(NCHUNK):
        gj = gidx_v.at[j]
        for t in range(CHUNK // LANES):
            v = idf_v[pl.ds(j * CHUNK + t * LANES, LANES)]
            gj[pl.ds(t * LANES, LANES)] = v - quarter(v) * QTR

    rbufs = (rbuf0, rbuf1)
    sels = (sel0, sel1)
    gsems = (gsem0, gsem1)

    def fire(slot):
        return pltpu.async_copy(p_hbm.at[gidx_v.at[slot]],
                                rbufs[slot % 2], gsems[slot % 2])

    def select(slot):
        rbuf = rbufs[slot % 2]
        sel = sels[slot % 2]
        for t in range(CHUNK // LANES):
            v = idf_v[pl.ds(slot * CHUNK + t * LANES, LANES)]
            q = quarter(v)
            khalf = (q >> 1) * HIDDEN
            hodd = (q & 1) == 1
            rvec = lane_iota + t * LANES

            @pl.loop(0, HIDDEN)
            def _(c):
                w = plsc.load_gather(rbuf, [rvec, khalf + c])
                lo = w & 0xFFFF
                hi = lax.shift_right_logical(w, 16)
                bits = jnp.where(hodd, hi, lo) << 16
                vals = lax.bitcast_convert_type(bits, jnp.float32)
                plsc.store_scatter(sel, [rvec, jnp.full((LANES,), c,
                                                        jnp.int32)], vals)

    pending = fire(0)
    writes = [None, None]
    for slot in range(NCHUNK):
        nxt = None
        if slot + 1 < NCHUNK:
            nxt = fire(slot + 1)
        pending.wait()
        if writes[slot % 2] is not None:
            writes[slot % 2].wait()
        select(slot)
        writes[slot % 2] = pltpu.async_copy(
            sels[slot % 2], out_hbm.at[pl.ds(base + slot * CHUNK, CHUNK)],
            wsem)
        pending = nxt
    for w in writes:
        if w is not None:
            w.wait()


BLK = 4096


def _mlp_body(u_ref, i_ref, w1_ref, b1_ref, w2_ref, b2_ref, w3_ref, b3_ref,
              o_ref):
    u = u_ref[...]
    it = i_ref[...]
    h = jnp.maximum(
        u @ w1_ref[:HIDDEN, :] + it @ w1_ref[HIDDEN:, :] + b1_ref[...], 0.0)
    h = jnp.maximum(h @ w2_ref[...] + b2_ref[...], 0.0)
    o_ref[...] = h @ w3_ref[...] + b3_ref[...]


_mlp = pl.pallas_call(
    _mlp_body,
    grid=(BATCH // BLK,),
    in_specs=[
        pl.BlockSpec((BLK, HIDDEN), lambda n: (n, 0)),
        pl.BlockSpec((BLK, HIDDEN), lambda n: (n, 0)),
        pl.BlockSpec((2 * HIDDEN, HIDDEN), lambda n: (0, 0)),
        pl.BlockSpec((1, HIDDEN), lambda n: (0, 0)),
        pl.BlockSpec((HIDDEN, HIDDEN // 2), lambda n: (0, 0)),
        pl.BlockSpec((1, HIDDEN // 2), lambda n: (0, 0)),
        pl.BlockSpec((HIDDEN // 2, HIDDEN // 4), lambda n: (0, 0)),
        pl.BlockSpec((1, HIDDEN // 4), lambda n: (0, 0)),
    ],
    out_specs=pl.BlockSpec((BLK, HIDDEN // 4), lambda n: (n, 0)),
    out_shape=jax.ShapeDtypeStruct((BATCH, HIDDEN // 4), jnp.float32),
    compiler_params=pltpu.CompilerParams(
        dimension_semantics=("arbitrary",)),
)


def kernel(user_id, item_id, user_table, item_table, W1, b1, W2, b2, W3, b3):
    uid = user_id.astype(jnp.int32).reshape(NW, BPW)
    iid = item_id.astype(jnp.int32).reshape(NW, BPW)
    tu = user_table.T
    ti = item_table.T
    pu = _tpack(tu, tu, tu, tu)
    u_emb = _sc_gather(uid, pu)
    pi = _tpack(ti, ti, ti, ti)
    i_emb = _sc_gather(iid, pi)
    return _mlp(u_emb, i_emb, W1, b1.reshape(1, -1), W2, b2.reshape(1, -1),
                W3, b3.reshape(1, -1))


# speedup vs baseline: 1.0655x; 1.0655x over previous
import functools

import jax
import jax.numpy as jnp
from jax import lax
from jax.experimental import pallas as pl
from jax.experimental.pallas import tpu as pltpu
from jax.experimental.pallas import tpu_sc as plsc

BATCH = 16384
HIDDEN = 64
NROWS = 1000000
NC = 2
NS = 16
NW = NC * NS
BPW = BATCH // NW
CHUNK = 128
NCHUNK = BPW // CHUNK


TBLK = 16384
QTR = 15 * TBLK
N2 = NROWS - 3 * QTR
TGRID = -(-N2 // TBLK)


def _pack2(a_ref, b_ref):
    a = a_ref[...].astype(jnp.bfloat16).T
    b = b_ref[...].astype(jnp.bfloat16).T
    ai = lax.bitcast_convert_type(a, jnp.uint16).astype(jnp.uint32)
    bi = lax.bitcast_convert_type(b, jnp.uint16).astype(jnp.uint32)
    return lax.bitcast_convert_type(ai | (bi << jnp.uint32(16)), jnp.int32)


def _tpack_body(t0_ref, t1_ref, t2_ref, t3_ref, o_ref):
    o_ref[:, :HIDDEN] = _pack2(t0_ref, t1_ref)
    o_ref[:, HIDDEN:] = _pack2(t2_ref, t3_ref)


_tpack = pl.pallas_call(
    _tpack_body,
    grid=(TGRID,),
    in_specs=[
        pl.BlockSpec((HIDDEN, TBLK), lambda n: (0, n)),
        pl.BlockSpec((HIDDEN, TBLK), lambda n: (0, n + 15)),
        pl.BlockSpec((HIDDEN, TBLK), lambda n: (0, n + 30)),
        pl.BlockSpec((HIDDEN, TBLK), lambda n: (0, n + 45)),
    ],
    out_specs=pl.BlockSpec((TBLK, 2 * HIDDEN), lambda n: (n, 0)),
    out_shape=jax.ShapeDtypeStruct((N2, 2 * HIDDEN), jnp.int32),
    compiler_params=pltpu.CompilerParams(
        dimension_semantics=("arbitrary",)),
)


_mesh = plsc.VectorSubcoreMesh(core_axis_name="c", subcore_axis_name="s")


@functools.partial(
    pl.kernel,
    mesh=_mesh,
    out_type=jax.ShapeDtypeStruct((BATCH, 2 * HIDDEN), jnp.int32),
    scratch_types=[
        pltpu.VMEM((NCHUNK, CHUNK), jnp.int32),
        pltpu.VMEM((CHUNK, 2 * HIDDEN), jnp.int32),
        pltpu.VMEM((CHUNK, 2 * HIDDEN), jnp.int32),
        pltpu.SemaphoreType.DMA,
        pltpu.SemaphoreType.DMA,
        pltpu.SemaphoreType.DMA,
    ],
    compiler_params=pltpu.CompilerParams(use_tc_tiling_on_sc=True,
                                         needs_layout_passes=False),
)
def _sc_gather(row_hbm, p_hbm, out_hbm, row_v, rbuf0, rbuf1,
               gsem0, gsem1, wsem):
    wid = lax.axis_index("s") * NC + lax.axis_index("c")
    base = wid * BPW
    pltpu.sync_copy(row_hbm.at[wid], row_v)

    chunks = [(p_hbm, row_v, out_hbm, j) for j in range(NCHUNK)]
    rbufs = (rbuf0, rbuf1)
    gsems = (gsem0, gsem1)

    def fire(slot):
        tblp, rref, _, j = chunks[slot]
        return pltpu.async_copy(tblp.at[rref.at[j]],
                                rbufs[slot % 2], gsems[slot % 2])

    pending = fire(0)
    writes = [None, None]
    for slot in range(len(chunks)):
        _, _, out, j = chunks[slot]
        nxt = None
        if slot + 1 < len(chunks):
            if writes[(slot + 1) % 2] is not None:
                writes[(slot + 1) % 2].wait()
                writes[(slot + 1) % 2] = None
            nxt = fire(slot + 1)
        pending.wait()
        writes[slot % 2] = pltpu.async_copy(
            rbufs[slot % 2], out.at[pl.ds(base + j * CHUNK, CHUNK)], wsem)
        pending = nxt
    writes[0].wait()
    writes[1].wait()


BLK = 4096


def _unpack_lo(w):
    return lax.bitcast_convert_type(
        (w & 0xFFFF).astype(jnp.uint16), jnp.bfloat16).astype(jnp.float32)


def _unpack_hi(w):
    u = lax.shift_right_logical(w.astype(jnp.uint32), jnp.uint32(16))
    return lax.bitcast_convert_type(
        u.astype(jnp.uint16), jnp.bfloat16).astype(jnp.float32)


def _qsel(g, id_col):
    w = jnp.where(id_col < 2 * QTR, g[:, :HIDDEN], g[:, HIDDEN:])
    odd = (id_col >= QTR) & (id_col < 2 * QTR) | (id_col >= 3 * QTR)
    return jnp.where(odd, _unpack_hi(w), _unpack_lo(w))


def _mlp_body(u_ref, i_ref, up_ref, ip_ref, w1_ref, b1_ref, w2_ref, b2_ref,
              w3_ref, b3_ref, o_ref):
    u = _qsel(u_ref[...], up_ref[...])
    it = _qsel(i_ref[...], ip_ref[...])
    h = jnp.maximum(
        u @ w1_ref[:HIDDEN, :] + it @ w1_ref[HIDDEN:, :] + b1_ref[...], 0.0)
    h = jnp.maximum(h @ w2_ref[...] + b2_ref[...], 0.0)
    o_ref[...] = h @ w3_ref[...] + b3_ref[...]


_mlp = pl.pallas_call(
    _mlp_body,
    grid=(BATCH // BLK,),
    in_specs=[
        pl.BlockSpec((BLK, 2 * HIDDEN), lambda n: (n, 0)),
        pl.BlockSpec((BLK, 2 * HIDDEN), lambda n: (n, 0)),
        pl.BlockSpec((BLK, 1), lambda n: (n, 0)),
        pl.BlockSpec((BLK, 1), lambda n: (n, 0)),
        pl.BlockSpec((2 * HIDDEN, HIDDEN), lambda n: (0, 0)),
        pl.BlockSpec((1, HIDDEN), lambda n: (0, 0)),
        pl.BlockSpec((HIDDEN, HIDDEN // 2), lambda n: (0, 0)),
        pl.BlockSpec((1, HIDDEN // 2), lambda n: (0, 0)),
        pl.BlockSpec((HIDDEN // 2, HIDDEN // 4), lambda n: (0, 0)),
        pl.BlockSpec((1, HIDDEN // 4), lambda n: (0, 0)),
    ],
    out_specs=pl.BlockSpec((BLK, HIDDEN // 4), lambda n: (n, 0)),
    out_shape=jax.ShapeDtypeStruct((BATCH, HIDDEN // 4), jnp.float32),
    compiler_params=pltpu.CompilerParams(
        dimension_semantics=("arbitrary",)),
)


def kernel(user_id, item_id, user_table, item_table, W1, b1, W2, b2, W3, b3):
    uid = user_id.astype(jnp.int32)
    iid = item_id.astype(jnp.int32)
    tu = user_table.T
    ti = item_table.T
    urow = (uid - jnp.minimum(uid // QTR, 3) * QTR).reshape(NW, NCHUNK, CHUNK)
    irow = (iid - jnp.minimum(iid // QTR, 3) * QTR).reshape(NW, NCHUNK, CHUNK)
    pu = _tpack(tu, tu, tu, tu)
    ug = _sc_gather(urow, pu)
    pi = _tpack(ti, ti, ti, ti)
    ig = _sc_gather(irow, pi)
    return _mlp(ug, ig, uid.reshape(BATCH, 1), iid.reshape(BATCH, 1),
                W1, b1.reshape(1, -1), W2, b2.reshape(1, -1),
                W3, b3.reshape(1, -1))


# speedup vs baseline: 1.0742x; 1.0081x over previous
import functools

import jax
import jax.numpy as jnp
from jax import lax
from jax.experimental import pallas as pl
from jax.experimental.pallas import tpu as pltpu
from jax.experimental.pallas import tpu_sc as plsc

BATCH = 16384
HIDDEN = 64
NROWS = 1000000
NC = 2
NS = 16
NW = NC * NS
BPW = BATCH // NW
CHUNK = 128
NCHUNK = BPW // CHUNK


TBLK = 16384
QTR = 15 * TBLK
N2 = NROWS - 3 * QTR
TGRID = -(-N2 // TBLK)


def _pack2(a_ref, b_ref):
    a = a_ref[...].astype(jnp.bfloat16).T
    b = b_ref[...].astype(jnp.bfloat16).T
    ai = lax.bitcast_convert_type(a, jnp.uint16).astype(jnp.uint32)
    bi = lax.bitcast_convert_type(b, jnp.uint16).astype(jnp.uint32)
    return lax.bitcast_convert_type(ai | (bi << jnp.uint32(16)), jnp.int32)


def _tpack_body(t0_ref, t1_ref, t2_ref, t3_ref, o_ref):
    o_ref[:, :HIDDEN] = _pack2(t0_ref, t1_ref)
    o_ref[:, HIDDEN:] = _pack2(t2_ref, t3_ref)


_tpack = pl.pallas_call(
    _tpack_body,
    grid=(TGRID,),
    in_specs=[
        pl.BlockSpec((HIDDEN, TBLK), lambda n: (0, n)),
        pl.BlockSpec((HIDDEN, TBLK), lambda n: (0, n + 15)),
        pl.BlockSpec((HIDDEN, TBLK), lambda n: (0, n + 30)),
        pl.BlockSpec((HIDDEN, TBLK), lambda n: (0, n + 45)),
    ],
    out_specs=pl.BlockSpec((TBLK, 2 * HIDDEN), lambda n: (n, 0)),
    out_shape=jax.ShapeDtypeStruct((N2, 2 * HIDDEN), jnp.int32),
    compiler_params=pltpu.CompilerParams(
        dimension_semantics=("arbitrary",)),
)


_mesh = plsc.VectorSubcoreMesh(core_axis_name="c", subcore_axis_name="s")


@functools.partial(
    pl.kernel,
    mesh=_mesh,
    out_type=jax.ShapeDtypeStruct((BATCH, 2 * HIDDEN), jnp.int32),
    scratch_types=[
        pltpu.VMEM((NCHUNK, CHUNK), jnp.int32),
        pltpu.VMEM((CHUNK, 2 * HIDDEN), jnp.int32),
        pltpu.VMEM((CHUNK, 2 * HIDDEN), jnp.int32),
        pltpu.SemaphoreType.DMA,
        pltpu.SemaphoreType.DMA,
        pltpu.SemaphoreType.DMA,
    ],
    compiler_params=pltpu.CompilerParams(use_tc_tiling_on_sc=True,
                                         needs_layout_passes=False),
)
def _sc_gather(row_hbm, p_hbm, out_hbm, row_v, rbuf0, rbuf1,
               gsem0, gsem1, wsem):
    wid = lax.axis_index("s") * NC + lax.axis_index("c")
    base = wid * BPW
    pltpu.sync_copy(row_hbm.at[wid], row_v)

    chunks = [(p_hbm, row_v, out_hbm, j) for j in range(NCHUNK)]
    rbufs = (rbuf0, rbuf1)
    gsems = (gsem0, gsem1)

    def fire(slot):
        tblp, rref, _, j = chunks[slot]
        return pltpu.async_copy(tblp.at[rref.at[j]],
                                rbufs[slot % 2], gsems[slot % 2])

    pending = fire(0)
    writes = [None, None]
    for slot in range(len(chunks)):
        _, _, out, j = chunks[slot]
        nxt = None
        if slot + 1 < len(chunks):
            if writes[(slot + 1) % 2] is not None:
                writes[(slot + 1) % 2].wait()
                writes[(slot + 1) % 2] = None
            nxt = fire(slot + 1)
        pending.wait()
        writes[slot % 2] = pltpu.async_copy(
            rbufs[slot % 2], out.at[pl.ds(base + j * CHUNK, CHUNK)], wsem)
        pending = nxt
    writes[0].wait()
    writes[1].wait()


BLK = 8192


def _unpack_lo(w):
    return lax.bitcast_convert_type(
        (w & 0xFFFF).astype(jnp.uint16), jnp.bfloat16).astype(jnp.float32)


def _unpack_hi(w):
    u = lax.shift_right_logical(w.astype(jnp.uint32), jnp.uint32(16))
    return lax.bitcast_convert_type(
        u.astype(jnp.uint16), jnp.bfloat16).astype(jnp.float32)


def _qsel(g, id_col):
    w = jnp.where(id_col < 2 * QTR, g[:, :HIDDEN], g[:, HIDDEN:])
    odd = (id_col >= QTR) & (id_col < 2 * QTR) | (id_col >= 3 * QTR)
    return jnp.where(odd, _unpack_hi(w), _unpack_lo(w))


def _mlp_body(u_ref, i_ref, ids_ref, w1_ref, b1_ref, w2_ref, b2_ref,
              w3_ref, b3_ref, o_ref):
    u = _qsel(u_ref[...], ids_ref[:, :1])
    it = _qsel(i_ref[...], ids_ref[:, 1:])
    h = jnp.maximum(
        u @ w1_ref[:HIDDEN, :] + it @ w1_ref[HIDDEN:, :] + b1_ref[...], 0.0)
    h = jnp.maximum(h @ w2_ref[...] + b2_ref[...], 0.0)
    o_ref[...] = h @ w3_ref[...] + b3_ref[...]


_mlp = pl.pallas_call(
    _mlp_body,
    grid=(BATCH // BLK,),
    in_specs=[
        pl.BlockSpec((BLK, 2 * HIDDEN), lambda n: (n, 0)),
        pl.BlockSpec((BLK, 2 * HIDDEN), lambda n: (n, 0)),
        pl.BlockSpec((BLK, 2), lambda n: (n, 0)),
        pl.BlockSpec((2 * HIDDEN, HIDDEN), lambda n: (0, 0)),
        pl.BlockSpec((1, HIDDEN), lambda n: (0, 0)),
        pl.BlockSpec((HIDDEN, HIDDEN // 2), lambda n: (0, 0)),
        pl.BlockSpec((1, HIDDEN // 2), lambda n: (0, 0)),
        pl.BlockSpec((HIDDEN // 2, HIDDEN // 4), lambda n: (0, 0)),
        pl.BlockSpec((1, HIDDEN // 4), lambda n: (0, 0)),
    ],
    out_specs=pl.BlockSpec((BLK, HIDDEN // 4), lambda n: (n, 0)),
    out_shape=jax.ShapeDtypeStruct((BATCH, HIDDEN // 4), jnp.float32),
    compiler_params=pltpu.CompilerParams(
        dimension_semantics=("arbitrary",)),
)


def kernel(user_id, item_id, user_table, item_table, W1, b1, W2, b2, W3, b3):
    uid = user_id.astype(jnp.int32)
    iid = item_id.astype(jnp.int32)
    tu = user_table.T
    ti = item_table.T
    urow = (uid - jnp.minimum(uid // QTR, 3) * QTR).reshape(NW, NCHUNK, CHUNK)
    irow = (iid - jnp.minimum(iid // QTR, 3) * QTR).reshape(NW, NCHUNK, CHUNK)
    pu = _tpack(tu, tu, tu, tu)
    ug = _sc_gather(urow, pu)
    pi = _tpack(ti, ti, ti, ti)
    ig = _sc_gather(irow, pi)
    ids2 = jnp.stack([uid, iid], axis=1)
    return _mlp(ug, ig, ids2, W1, b1.reshape(1, -1), W2, b2.reshape(1, -1),
                W3, b3.reshape(1, -1))


# speedup vs baseline: 1.0812x; 1.0066x over previous
import functools

import jax
import jax.numpy as jnp
from jax import lax
from jax.experimental import pallas as pl
from jax.experimental.pallas import tpu as pltpu
from jax.experimental.pallas import tpu_sc as plsc

BATCH = 16384
HIDDEN = 64
NROWS = 1000000
NC = 2
NS = 16
NW = NC * NS
BPW = BATCH // NW
CHUNK = 128
NCHUNK = BPW // CHUNK


TBLK = 16384
QTR = 15 * TBLK
N2 = NROWS - 3 * QTR
TGRID = -(-N2 // TBLK)


def _pack2(a_ref, b_ref):
    a = a_ref[...].astype(jnp.bfloat16)
    b = b_ref[...].astype(jnp.bfloat16)
    ai = lax.bitcast_convert_type(a, jnp.uint16).astype(jnp.uint32)
    bi = lax.bitcast_convert_type(b, jnp.uint16).astype(jnp.uint32)
    w = lax.bitcast_convert_type(ai | (bi << jnp.uint32(16)), jnp.int32)
    return w.T


def _tpack_body(t0_ref, t1_ref, t2_ref, t3_ref, o_ref):
    o_ref[:, :HIDDEN] = _pack2(t0_ref, t1_ref)
    o_ref[:, HIDDEN:] = _pack2(t2_ref, t3_ref)


_tpack = pl.pallas_call(
    _tpack_body,
    grid=(TGRID,),
    in_specs=[
        pl.BlockSpec((HIDDEN, TBLK), lambda n: (0, n)),
        pl.BlockSpec((HIDDEN, TBLK), lambda n: (0, n + 15)),
        pl.BlockSpec((HIDDEN, TBLK), lambda n: (0, n + 30)),
        pl.BlockSpec((HIDDEN, TBLK), lambda n: (0, n + 45)),
    ],
    out_specs=pl.BlockSpec((TBLK, 2 * HIDDEN), lambda n: (n, 0)),
    out_shape=jax.ShapeDtypeStruct((N2, 2 * HIDDEN), jnp.int32),
    compiler_params=pltpu.CompilerParams(
        dimension_semantics=("arbitrary",)),
)


_mesh = plsc.VectorSubcoreMesh(core_axis_name="c", subcore_axis_name="s")


@functools.partial(
    pl.kernel,
    mesh=_mesh,
    out_type=jax.ShapeDtypeStruct((BATCH, 2 * HIDDEN), jnp.int32),
    scratch_types=[
        pltpu.VMEM((NCHUNK, CHUNK), jnp.int32),
        pltpu.VMEM((CHUNK, 2 * HIDDEN), jnp.int32),
        pltpu.VMEM((CHUNK, 2 * HIDDEN), jnp.int32),
        pltpu.SemaphoreType.DMA,
        pltpu.SemaphoreType.DMA,
        pltpu.SemaphoreType.DMA,
    ],
    compiler_params=pltpu.CompilerParams(use_tc_tiling_on_sc=True,
                                         needs_layout_passes=False),
)
def _sc_gather(row_hbm, p_hbm, out_hbm, row_v, rbuf0, rbuf1,
               gsem0, gsem1, wsem):
    wid = lax.axis_index("s") * NC + lax.axis_index("c")
    base = wid * BPW
    pltpu.sync_copy(row_hbm.at[wid], row_v)

    chunks = [(p_hbm, row_v, out_hbm, j) for j in range(NCHUNK)]
    rbufs = (rbuf0, rbuf1)
    gsems = (gsem0, gsem1)

    def fire(slot):
        tblp, rref, _, j = chunks[slot]
        return pltpu.async_copy(tblp.at[rref.at[j]],
                                rbufs[slot % 2], gsems[slot % 2])

    pending = fire(0)
    writes = [None, None]
    for slot in range(len(chunks)):
        _, _, out, j = chunks[slot]
        nxt = None
        if slot + 1 < len(chunks):
            if writes[(slot + 1) % 2] is not None:
                writes[(slot + 1) % 2].wait()
                writes[(slot + 1) % 2] = None
            nxt = fire(slot + 1)
        pending.wait()
        writes[slot % 2] = pltpu.async_copy(
            rbufs[slot % 2], out.at[pl.ds(base + j * CHUNK, CHUNK)], wsem)
        pending = nxt
    writes[0].wait()
    writes[1].wait()


BLK = 8192


def _unpack_lo(w):
    return lax.bitcast_convert_type(
        (w & 0xFFFF).astype(jnp.uint16), jnp.bfloat16).astype(jnp.float32)


def _unpack_hi(w):
    u = lax.shift_right_logical(w.astype(jnp.uint32), jnp.uint32(16))
    return lax.bitcast_convert_type(
        u.astype(jnp.uint16), jnp.bfloat16).astype(jnp.float32)


def _qsel(g, id_col):
    w = jnp.where(id_col < 2 * QTR, g[:, :HIDDEN], g[:, HIDDEN:])
    odd = (id_col >= QTR) & (id_col < 2 * QTR) | (id_col >= 3 * QTR)
    return jnp.where(odd, _unpack_hi(w), _unpack_lo(w))


def _mlp_body(u_ref, i_ref, ids_ref, w1_ref, b1_ref, w2_ref, b2_ref,
              w3_ref, b3_ref, o_ref):
    u = _qsel(u_ref[...], ids_ref[:, :1])
    it = _qsel(i_ref[...], ids_ref[:, 1:])
    h = jnp.maximum(
        u @ w1_ref[:HIDDEN, :] + it @ w1_ref[HIDDEN:, :] + b1_ref[...], 0.0)
    h = jnp.maximum(h @ w2_ref[...] + b2_ref[...], 0.0)
    o_ref[...] = h @ w3_ref[...] + b3_ref[...]


_mlp = pl.pallas_call(
    _mlp_body,
    grid=(BATCH // BLK,),
    in_specs=[
        pl.BlockSpec((BLK, 2 * HIDDEN), lambda n: (n, 0)),
        pl.BlockSpec((BLK, 2 * HIDDEN), lambda n: (n, 0)),
        pl.BlockSpec((BLK, 2), lambda n: (n, 0)),
        pl.BlockSpec((2 * HIDDEN, HIDDEN), lambda n: (0, 0)),
        pl.BlockSpec((1, HIDDEN), lambda n: (0, 0)),
        pl.BlockSpec((HIDDEN, HIDDEN // 2), lambda n: (0, 0)),
        pl.BlockSpec((1, HIDDEN // 2), lambda n: (0, 0)),
        pl.BlockSpec((HIDDEN // 2, HIDDEN // 4), lambda n: (0, 0)),
        pl.BlockSpec((1, HIDDEN // 4), lambda n: (0, 0)),
    ],
    out_specs=pl.BlockSpec((BLK, HIDDEN // 4), lambda n: (n, 0)),
    out_shape=jax.ShapeDtypeStruct((BATCH, HIDDEN // 4), jnp.float32),
    compiler_params=pltpu.CompilerParams(
        dimension_semantics=("arbitrary",)),
)


def kernel(user_id, item_id, user_table, item_table, W1, b1, W2, b2, W3, b3):
    uid = user_id.astype(jnp.int32)
    iid = item_id.astype(jnp.int32)
    tu = user_table.T
    ti = item_table.T
    urow = (uid - jnp.minimum(uid // QTR, 3) * QTR).reshape(NW, NCHUNK, CHUNK)
    irow = (iid - jnp.minimum(iid // QTR, 3) * QTR).reshape(NW, NCHUNK, CHUNK)
    pu = _tpack(tu, tu, tu, tu)
    ug = _sc_gather(urow, pu)
    pi = _tpack(ti, ti, ti, ti)
    ig = _sc_gather(irow, pi)
    ids2 = jnp.stack([uid, iid], axis=1)
    return _mlp(ug, ig, ids2, W1, b1.reshape(1, -1), W2, b2.reshape(1, -1),
                W3, b3.reshape(1, -1))
